# baseline (speedup 1.0000x reference)
import functools

import jax
import jax.numpy as jnp
from jax.experimental import pallas as pl

F32 = jnp.float32


def _mish(x):
    sp = jnp.where(x > 20.0, x, jnp.log1p(jnp.exp(jnp.minimum(x, 20.0))))
    return x * jnp.tanh(sp)


def _tree_body(x_ref, idx_ref, w1, b1r, w2, b2r, w3, b3r, w4, b4r, out_ref):
    for nb in range(x_ref.shape[0]):
        _one_tree(x_ref, idx_ref, (w1, b1r, w2, b2r, w3, b3r, w4, b4r), out_ref, nb)


def _one_tree(x_ref, idx_ref, wbs, out_ref, nb):
    (w1, b1r, w2, b2r, w3, b3r, w4, b4r) = wbs
    x = x_ref[nb]
    ids = idx_ref[nb]
    T = x.shape[1]
    H = T // 2
    col0 = jax.lax.broadcasted_iota(jnp.int32, (1, T), 1) == 0

    for (wr, br) in ((w1, b1r), (w2, b2r), (w3, b3r), (w4, b4r)):
        w = wr[...]
        cout = w.shape[0] // 3
        xb = x.astype(jnp.bfloat16)
        ystack = jnp.dot(w, xb, preferred_element_type=F32)
        acc = None
        for k in range(3):
            yk = ystack[k * cout:(k + 1) * cout, :]
            idx = jnp.broadcast_to(ids[k:k + 1, :], (cout, T))
            glo = jnp.take_along_axis(yk[:, :H], jnp.minimum(idx, H - 1), axis=1)
            ghi = jnp.take_along_axis(yk[:, H:], jnp.maximum(idx - H, 0), axis=1)
            t = jnp.where(idx < H, glo, ghi)
            acc = t if acc is None else acc + t
        out = acc + br[...]
        out = jnp.where(col0, 0.0, out)
        n = out.size
        m = jnp.mean(out)
        ss = jnp.sum((out - m) ** 2)
        std = jnp.sqrt(ss / (n - 1))
        x = _mish((out - m) / (std + 1e-5))
    out_ref[nb, 0, :] = jnp.max(x, axis=1)


def _final_body(p_ref, wt_ref, lb_ref, g_ref, b_ref, out_ref):
    y0 = jnp.dot(p_ref[...].astype(jnp.bfloat16), wt_ref[...].astype(jnp.bfloat16),
                 preferred_element_type=F32) + lb_ref[...]
    mean = jnp.mean(y0, axis=0, keepdims=True)
    var = jnp.mean((y0 - mean) ** 2, axis=0, keepdims=True)
    out_ref[...] = (y0 - mean) / jnp.sqrt(var + 1e-5) * g_ref[...] + b_ref[...]


@jax.jit
def kernel(trees_data, trees_indexes, W1, b1, W2, b2, W3, b3, W4, b4, lin_W, lin_b, bn_g, bn_b):
    B, C0, T = trees_data.shape

    idx = trees_indexes.reshape(B, T - 1, 3).transpose(0, 2, 1).astype(jnp.int32)
    sent = jnp.zeros((B, 3, 1), jnp.int32)
    idxp = jnp.concatenate([sent, idx], axis=2)

    ws = [jnp.transpose(W, (2, 0, 1)).reshape(3 * W.shape[0], W.shape[1]).astype(jnp.bfloat16)
          for W in (W1, W2, W3, W4)]
    bs = [b.reshape(-1, 1) for b in (b1, b2, b3, b4)]

    def full(s):
        return pl.BlockSpec(s, lambda *_: (0,) * len(s))

    NB = 4
    in_specs = [
        pl.BlockSpec((NB, C0, T), lambda i: (i, 0, 0)),
        pl.BlockSpec((NB, 3, T), lambda i: (i, 0, 0)),
    ]
    for w, b in zip(ws, bs):
        in_specs.append(full(w.shape))
        in_specs.append(full(b.shape))

    pooled = pl.pallas_call(
        _tree_body,
        grid=(B // NB,),
        in_specs=in_specs,
        out_specs=pl.BlockSpec((NB, 1, ws[-1].shape[0] // 3), lambda i: (i, 0, 0)),
        out_shape=jax.ShapeDtypeStruct((B, 1, ws[-1].shape[0] // 3), F32),
    )(trees_data, idxp, ws[0], bs[0], ws[1], bs[1], ws[2], bs[2], ws[3], bs[3])
    pooled = pooled.reshape(B, ws[-1].shape[0] // 3)

    Z = lin_W.shape[0]
    y = pl.pallas_call(
        _final_body,
        in_specs=[
            pl.BlockSpec(pooled.shape, lambda: (0, 0)),
            pl.BlockSpec((lin_W.shape[1], Z), lambda: (0, 0)),
            pl.BlockSpec((1, Z), lambda: (0, 0)),
            pl.BlockSpec((1, Z), lambda: (0, 0)),
            pl.BlockSpec((1, Z), lambda: (0, 0)),
        ],
        out_specs=pl.BlockSpec((B, Z), lambda: (0, 0)),
        out_shape=jax.ShapeDtypeStruct((B, Z), F32),
    )(pooled, lin_W.T, lin_b.reshape(1, -1), bn_g.reshape(1, -1), bn_b.reshape(1, -1))

    return (y, trees_indexes)


# speedup vs baseline: 1.7700x; 1.7700x over previous
import functools

import jax
import jax.numpy as jnp
from jax.experimental import pallas as pl

F32 = jnp.float32


def _mish(x):
    u = jnp.exp(jnp.minimum(x, 20.0))
    num = u * (u + 2.0)
    y = x * num / (num + 2.0)
    return jnp.where(x > 20.0, x, y)


def _tree_body(x_ref, idx_ref, w1, b1r, w2, b2r, w3, b3r, w4, b4r, out_ref):
    NB = x_ref.shape[0]
    T = x_ref.shape[2]
    H = T // 2
    xcat = jnp.concatenate([x_ref[nb] for nb in range(NB)], axis=1)
    col0 = jax.lax.broadcasted_iota(jnp.int32, (1, T), 1) == 0

    for (wr, br) in ((w1, b1r), (w2, b2r), (w3, b3r), (w4, b4r)):
        w = wr[...]
        cout = w.shape[0] // 3
        xb = xcat.astype(jnp.bfloat16)
        ystack = jnp.dot(w, xb, preferred_element_type=F32)
        outs = []
        for nb in range(NB):
            ids = idx_ref[nb]
            acc = None
            for k in range(3):
                yk = ystack[k * cout:(k + 1) * cout, nb * T:(nb + 1) * T]
                idx = jnp.broadcast_to(ids[k:k + 1, :], (cout, T))
                glo = jnp.take_along_axis(yk[:, :H], jnp.minimum(idx, H - 1), axis=1)
                ghi = jnp.take_along_axis(yk[:, H:], jnp.maximum(idx - H, 0), axis=1)
                t = jnp.where(idx < H, glo, ghi)
                acc = t if acc is None else acc + t
            out = acc + br[...]
            out = jnp.where(col0, 0.0, out)
            n = out.size
            s1 = jnp.sum(out)
            s2 = jnp.sum(out * out)
            m = s1 / n
            std = jnp.sqrt((s2 - s1 * m) / (n - 1))
            outs.append((out - m) * (1.0 / (std + 1e-5)))
        xcat = _mish(jnp.concatenate(outs, axis=1))
    for nb in range(NB):
        out_ref[nb, 0, :] = jnp.max(xcat[:, nb * T:(nb + 1) * T], axis=1)


def _final_body(p_ref, wt_ref, lb_ref, g_ref, b_ref, out_ref):
    y0 = jnp.dot(p_ref[...].astype(jnp.bfloat16), wt_ref[...].astype(jnp.bfloat16),
                 preferred_element_type=F32) + lb_ref[...]
    mean = jnp.mean(y0, axis=0, keepdims=True)
    var = jnp.mean((y0 - mean) ** 2, axis=0, keepdims=True)
    out_ref[...] = (y0 - mean) / jnp.sqrt(var + 1e-5) * g_ref[...] + b_ref[...]


@jax.jit
def kernel(trees_data, trees_indexes, W1, b1, W2, b2, W3, b3, W4, b4, lin_W, lin_b, bn_g, bn_b):
    B, C0, T = trees_data.shape

    idx = trees_indexes.reshape(B, T - 1, 3).transpose(0, 2, 1).astype(jnp.int32)
    sent = jnp.zeros((B, 3, 1), jnp.int32)
    idxp = jnp.concatenate([sent, idx], axis=2)

    ws = [jnp.transpose(W, (2, 0, 1)).reshape(3 * W.shape[0], W.shape[1]).astype(jnp.bfloat16)
          for W in (W1, W2, W3, W4)]
    bs = [b.reshape(-1, 1) for b in (b1, b2, b3, b4)]

    def full(s):
        return pl.BlockSpec(s, lambda *_: (0,) * len(s))

    NB = 4
    in_specs = [
        pl.BlockSpec((NB, C0, T), lambda i: (i, 0, 0)),
        pl.BlockSpec((NB, 3, T), lambda i: (i, 0, 0)),
    ]
    for w, b in zip(ws, bs):
        in_specs.append(full(w.shape))
        in_specs.append(full(b.shape))

    pooled = pl.pallas_call(
        _tree_body,
        grid=(B // NB,),
        in_specs=in_specs,
        out_specs=pl.BlockSpec((NB, 1, ws[-1].shape[0] // 3), lambda i: (i, 0, 0)),
        out_shape=jax.ShapeDtypeStruct((B, 1, ws[-1].shape[0] // 3), F32),
    )(trees_data, idxp, ws[0], bs[0], ws[1], bs[1], ws[2], bs[2], ws[3], bs[3])
    pooled = pooled.reshape(B, ws[-1].shape[0] // 3)

    Z = lin_W.shape[0]
    y = pl.pallas_call(
        _final_body,
        in_specs=[
            pl.BlockSpec(pooled.shape, lambda: (0, 0)),
            pl.BlockSpec((lin_W.shape[1], Z), lambda: (0, 0)),
            pl.BlockSpec((1, Z), lambda: (0, 0)),
            pl.BlockSpec((1, Z), lambda: (0, 0)),
            pl.BlockSpec((1, Z), lambda: (0, 0)),
        ],
        out_specs=pl.BlockSpec((B, Z), lambda: (0, 0)),
        out_shape=jax.ShapeDtypeStruct((B, Z), F32),
    )(pooled, lin_W.T, lin_b.reshape(1, -1), bn_g.reshape(1, -1), bn_b.reshape(1, -1))

    return (y, trees_indexes)


# speedup vs baseline: 1.8554x; 1.0482x over previous
import functools

import jax
import jax.numpy as jnp
from jax.experimental import pallas as pl

F32 = jnp.float32


def _mish(x):
    u = jnp.exp(jnp.minimum(x, 20.0))
    num = u * (u + 2.0)
    y = x * num / (num + 2.0)
    return jnp.where(x > 20.0, x, y)


def _tree_body(x_ref, idx_ref, w1, b1r, w2, b2r, w3, b3r, w4, b4r, out_ref):
    NB = x_ref.shape[0]
    T = x_ref.shape[2]
    H = T // 2
    xcat = jnp.concatenate([x_ref[nb] for nb in range(NB)], axis=1)
    col0 = jax.lax.broadcasted_iota(jnp.int32, (1, T), 1) == 0

    for (wr, br) in ((w1, b1r), (w2, b2r), (w3, b3r), (w4, b4r)):
        w = wr[...]
        cout = w.shape[0] // 3
        xb = xcat.astype(jnp.bfloat16)
        ystack = jnp.dot(w, xb, preferred_element_type=F32)
        outs = []
        for nb in range(NB):
            ids = idx_ref[nb]
            acc = None
            for k in range(3):
                yk = ystack[k * cout:(k + 1) * cout, nb * T:(nb + 1) * T]
                idx = jnp.broadcast_to(ids[k:k + 1, :], (cout, T))
                glo = jnp.take_along_axis(yk[:, :H], jnp.minimum(idx, H - 1), axis=1)
                ghi = jnp.take_along_axis(yk[:, H:], jnp.maximum(idx - H, 0), axis=1)
                t = jnp.where(idx < H, glo, ghi)
                acc = t if acc is None else acc + t
            out = acc + br[...]
            out = jnp.where(col0, 0.0, out)
            n = out.size
            s1 = jnp.sum(out)
            s2 = jnp.sum(out * out)
            m = s1 / n
            std = jnp.sqrt((s2 - s1 * m) / (n - 1))
            outs.append((out - m) * (1.0 / (std + 1e-5)))
        xcat = _mish(jnp.concatenate(outs, axis=1))
    for nb in range(NB):
        out_ref[nb, 0, :] = jnp.max(xcat[:, nb * T:(nb + 1) * T], axis=1)


def _final_body(p_ref, wt_ref, lb_ref, g_ref, b_ref, out_ref):
    y0 = jnp.dot(p_ref[...].astype(jnp.bfloat16), wt_ref[...].astype(jnp.bfloat16),
                 preferred_element_type=F32) + lb_ref[...]
    mean = jnp.mean(y0, axis=0, keepdims=True)
    var = jnp.mean((y0 - mean) ** 2, axis=0, keepdims=True)
    out_ref[...] = (y0 - mean) / jnp.sqrt(var + 1e-5) * g_ref[...] + b_ref[...]


@jax.jit
def kernel(trees_data, trees_indexes, W1, b1, W2, b2, W3, b3, W4, b4, lin_W, lin_b, bn_g, bn_b):
    B, C0, T = trees_data.shape

    idx = trees_indexes.reshape(B, T - 1, 3).transpose(0, 2, 1).astype(jnp.int32)
    sent = jnp.zeros((B, 3, 1), jnp.int32)
    idxp = jnp.concatenate([sent, idx], axis=2)

    ws = [jnp.transpose(W, (2, 0, 1)).reshape(3 * W.shape[0], W.shape[1]).astype(jnp.bfloat16)
          for W in (W1, W2, W3, W4)]
    bs = [b.reshape(-1, 1) for b in (b1, b2, b3, b4)]

    def full(s):
        return pl.BlockSpec(s, lambda *_: (0,) * len(s))

    NB = 8
    in_specs = [
        pl.BlockSpec((NB, C0, T), lambda i: (i, 0, 0)),
        pl.BlockSpec((NB, 3, T), lambda i: (i, 0, 0)),
    ]
    for w, b in zip(ws, bs):
        in_specs.append(full(w.shape))
        in_specs.append(full(b.shape))

    pooled = pl.pallas_call(
        _tree_body,
        grid=(B // NB,),
        in_specs=in_specs,
        out_specs=pl.BlockSpec((NB, 1, ws[-1].shape[0] // 3), lambda i: (i, 0, 0)),
        out_shape=jax.ShapeDtypeStruct((B, 1, ws[-1].shape[0] // 3), F32),
    )(trees_data, idxp, ws[0], bs[0], ws[1], bs[1], ws[2], bs[2], ws[3], bs[3])
    pooled = pooled.reshape(B, ws[-1].shape[0] // 3)

    Z = lin_W.shape[0]
    y = pl.pallas_call(
        _final_body,
        in_specs=[
            pl.BlockSpec(pooled.shape, lambda: (0, 0)),
            pl.BlockSpec((lin_W.shape[1], Z), lambda: (0, 0)),
            pl.BlockSpec((1, Z), lambda: (0, 0)),
            pl.BlockSpec((1, Z), lambda: (0, 0)),
            pl.BlockSpec((1, Z), lambda: (0, 0)),
        ],
        out_specs=pl.BlockSpec((B, Z), lambda: (0, 0)),
        out_shape=jax.ShapeDtypeStruct((B, Z), F32),
    )(pooled, lin_W.T, lin_b.reshape(1, -1), bn_g.reshape(1, -1), bn_b.reshape(1, -1))

    return (y, trees_indexes)
